# baseline (speedup 1.0000x reference)
import functools

import jax
import jax.numpy as jnp
from jax import lax
from jax.experimental import pallas as pl
from jax.experimental.pallas import tpu as pltpu
from jax.experimental.pallas import tpu_sc as plsc

B, S, V = 8, 32, 100000
ROWS = B * S
LANES = 16
NUM_WORKERS = 32
ROWS_PER_WORKER = ROWS // NUM_WORKERS
NUM_CHUNKS = 2
CHUNK = V // NUM_CHUNKS
GPC = CHUNK // LANES


def _sc_body(x3_hbm, lab_hbm, out_hbm,
             buf0, buf1, vbuf, lab_v, res_v, sem0, sem1):
    cid = lax.axis_index("c")
    sid = lax.axis_index("s")
    wid = sid * 2 + cid
    base = wid * ROWS_PER_WORKER
    lane = lax.iota(jnp.int32, LANES)
    bufs = (buf0, buf1)
    sems = (sem0, sem1)
    pltpu.sync_copy(lab_hbm, lab_v)
    pltpu.async_copy(x3_hbm.at[base, 0], buf0, sem0)

    acc = jnp.float32(0.0)
    for r in range(ROWS_PER_WORKER):
        row = base + r
        g0 = row // LANES
        rl = row - g0 * LANES
        lvec = lab_v[pl.ds(g0 * LANES, LANES)]
        lab = jnp.sum(jnp.where(lane == rl, lvec, 0))
        g_lab = lab // LANES
        rloc = lab - g_lab * LANES
        c_star = g_lab // GPC
        off = (g_lab - c_star * GPC) * LANES
        pltpu.sync_copy(x3_hbm.at[row, c_star, pl.ds(off, LANES)], vbuf)
        ab = vbuf[...].astype(jnp.int32)
        v_splat = jnp.broadcast_to(jnp.sum(jnp.where(lane == rloc, ab, 0)), (LANES,))
        mb = (ab > v_splat) | ((ab == v_splat) & (lane < rloc))
        zeros = jnp.zeros((LANES,), jnp.int32)
        cnt = mb.astype(jnp.int32)
        for c in range(NUM_CHUNKS):
            if c + 1 < NUM_CHUNKS:
                pltpu.async_copy(x3_hbm.at[row, c + 1], bufs[(c + 1) % 2],
                                 sems[(c + 1) % 2])
            elif r + 1 < ROWS_PER_WORKER:
                pltpu.async_copy(x3_hbm.at[row + 1, 0], bufs[0], sems[0])
            pltpu.make_async_copy(x3_hbm.at[row, c], bufs[c % 2],
                                  sems[c % 2]).wait()
            buf = bufs[c % 2]
            n_lo = jnp.clip(g_lab - c * GPC, 0, GPC)
            n_hi = jnp.clip(g_lab + 1 - c * GPC, 0, GPC)

            def body_lo(g, cc, _buf=buf):
                a = _buf[pl.ds(g, LANES)].astype(jnp.int32)
                return cc + (a >= v_splat).astype(jnp.int32)

            def body_hi(g, cc, _buf=buf):
                a = _buf[pl.ds(g, LANES)].astype(jnp.int32)
                return cc + (a > v_splat).astype(jnp.int32)

            cnt_lo = plsc.parallel_loop(
                0, n_lo * LANES, LANES, unroll=8, carry=zeros)(body_lo)
            cnt_hi = plsc.parallel_loop(
                n_hi * LANES, CHUNK, LANES, unroll=8, carry=zeros)(body_hi)
            cnt = cnt + cnt_lo + cnt_hi
        total = jnp.sum(cnt)
        acc = acc + jnp.where(total <= 4, jnp.float32(1.0 / ROWS), jnp.float32(0.0))
    res_v[...] = jnp.broadcast_to(acc, (LANES,))
    pltpu.sync_copy(res_v, out_hbm.at[wid])


_sc_count = functools.partial(
    pl.kernel,
    out_type=jax.ShapeDtypeStruct((NUM_WORKERS, LANES), jnp.float32),
    mesh=plsc.VectorSubcoreMesh(core_axis_name="c", subcore_axis_name="s"),
    scratch_types=[
        pltpu.VMEM((CHUNK,), jnp.float32),
        pltpu.VMEM((CHUNK,), jnp.float32),
        pltpu.VMEM((LANES,), jnp.float32),
        pltpu.VMEM((ROWS,), jnp.int32),
        pltpu.VMEM((LANES,), jnp.float32),
        pltpu.SemaphoreType.DMA,
        pltpu.SemaphoreType.DMA,
    ],
    compiler_params=pltpu.CompilerParams(needs_layout_passes=False),
)(_sc_body)


def _tc_combine(p_ref, o_ref):
    o_ref[0, 0] = jnp.sum(p_ref[...]) * jnp.float32(1.0 / LANES)


def kernel(y_true, y_pred):
    labels = y_true.astype(jnp.int32).reshape(ROWS)
    x3 = y_pred.reshape(ROWS, NUM_CHUNKS, CHUNK)
    partials = _sc_count(x3, labels)
    value2d = pl.pallas_call(
        _tc_combine,
        out_shape=jax.ShapeDtypeStruct((1, 1), jnp.float32),
        in_specs=[pl.BlockSpec(memory_space=pltpu.VMEM)],
        out_specs=pl.BlockSpec(memory_space=pltpu.SMEM),
    )(partials)
    return (y_pred, value2d.reshape(()))


# speedup vs baseline: 2.8107x; 2.8107x over previous
import functools

import jax
import jax.numpy as jnp
from jax import lax
from jax.experimental import pallas as pl
from jax.experimental.pallas import tpu as pltpu
from jax.experimental.pallas import tpu_sc as plsc

B, S, V = 8, 32, 100000
ROWS = B * S
LANES = 16
NUM_WORKERS = 32
ROWS_PER_WORKER = ROWS // NUM_WORKERS
NUM_GROUPS = V // LANES


def _prevfloat_pos(f):
    return plsc.bitcast(plsc.bitcast(f, jnp.int32) - 1, jnp.float32)


def _sc_body(x_hbm, lab_hbm, out_hbm, row_v, lab_v, res_v):
    cid = lax.axis_index("c")
    sid = lax.axis_index("s")
    wid = sid * 2 + cid
    base = wid * ROWS_PER_WORKER
    lane = lax.iota(jnp.int32, LANES)
    pltpu.sync_copy(lab_hbm, lab_v)
    acc = jnp.float32(0.0)
    for r in range(ROWS_PER_WORKER):
        row = base + r
        pltpu.sync_copy(x_hbm.at[row], row_v)
        g0 = row // LANES
        rl = row - g0 * LANES
        lvec = lab_v[pl.ds(g0 * LANES, LANES)]
        lab = jnp.sum(jnp.where(lane == rl, lvec, 0))
        g_lab = lab // LANES
        rloc = lab - g_lab * LANES
        ab = row_v[pl.ds(g_lab * LANES, LANES)].astype(jnp.int32)
        vi = jnp.broadcast_to(jnp.sum(jnp.where(lane == rloc, ab, 0)), (LANES,))
        c1 = vi.astype(jnp.float32)
        t_ge = jnp.where(vi > 0, _prevfloat_pos(c1), c1 - 1.0)
        c2 = (vi + 1).astype(jnp.float32)
        t_gt = jnp.where(vi + 1 > 0, _prevfloat_pos(c2), c2 - 1.0)
        mb = (ab > vi) | ((ab == vi) & (lane < rloc))
        cnt0 = mb.astype(jnp.int32)

        def body_lo(g, cc):
            return cc + (row_v[pl.ds(g, LANES)] > t_ge).astype(jnp.int32)

        def body_hi(g, cc):
            return cc + (row_v[pl.ds(g, LANES)] > t_gt).astype(jnp.int32)

        zeros = jnp.zeros((LANES,), jnp.int32)
        cnt_lo = plsc.parallel_loop(
            0, g_lab * LANES, LANES, unroll=8, carry=zeros)(body_lo)
        cnt_hi = plsc.parallel_loop(
            (g_lab + 1) * LANES, V, LANES, unroll=8, carry=zeros)(body_hi)
        total = jnp.sum(cnt0 + cnt_lo + cnt_hi)
        acc = acc + jnp.where(total <= 4, jnp.float32(1.0 / ROWS), jnp.float32(0.0))
    res_v[...] = jnp.broadcast_to(acc, (LANES,))
    pltpu.sync_copy(res_v, out_hbm.at[wid])


_sc_count = functools.partial(
    pl.kernel,
    out_type=jax.ShapeDtypeStruct((NUM_WORKERS, LANES), jnp.float32),
    mesh=plsc.VectorSubcoreMesh(core_axis_name="c", subcore_axis_name="s"),
    scratch_types=[
        pltpu.VMEM((V,), jnp.float32),
        pltpu.VMEM((ROWS,), jnp.int32),
        pltpu.VMEM((LANES,), jnp.float32),
    ],
    compiler_params=pltpu.CompilerParams(needs_layout_passes=False),
)(_sc_body)


def _tc_combine(p_ref, o_ref):
    o_ref[0, 0] = jnp.sum(p_ref[...]) * jnp.float32(1.0 / LANES)


def _tc_copy(x_ref, o_ref):
    o_ref[...] = x_ref[...]


def kernel(y_true, y_pred):
    labels = y_true.astype(jnp.int32).reshape(ROWS)
    x = y_pred.reshape(ROWS, V)
    partials = _sc_count(x, labels)
    y_out = pl.pallas_call(
        _tc_copy,
        grid=(B, 4),
        in_specs=[pl.BlockSpec((1, S // 4, V), lambda i, j: (i, j, 0))],
        out_specs=pl.BlockSpec((1, S // 4, V), lambda i, j: (i, j, 0)),
        out_shape=jax.ShapeDtypeStruct((B, S, V), jnp.float32),
    )(y_pred)
    value2d = pl.pallas_call(
        _tc_combine,
        out_shape=jax.ShapeDtypeStruct((1, 1), jnp.float32),
        in_specs=[pl.BlockSpec(memory_space=pltpu.VMEM)],
        out_specs=pl.BlockSpec(memory_space=pltpu.SMEM),
    )(partials)
    return (y_out, value2d.reshape(()))
